# baseline (speedup 1.0000x reference)
import functools

import jax
import jax.numpy as jnp
from jax import lax
from jax.experimental import pallas as pl
from jax.experimental.pallas import tpu as pltpu
from jax.experimental.pallas import tpu_sc as plsc

NUM_EXPERTS = 8
DIM = 1024
DIM4ROUTE = 128
HID = 4096
T = 4096
TB = 1024
BT = 256
PADDED_T = T + NUM_EXPERTS * BT
NUM_TILES = PADDED_T // BT

NC = 2
NS = 16
NW = NC * NS
TPW = T // NW
CH = 32
NCH = TPW // CH


def _routing_body(x_ref, wr_ref, norms_ref, sel_ref, rank_ref, counts_ref,
                  loss_ref, runc_ref, psum_ref):
    i = pl.program_id(0)

    @pl.when(i == 0)
    def _init():
        runc_ref[...] = jnp.zeros_like(runc_ref)
        psum_ref[...] = jnp.zeros_like(psum_ref)

    x = x_ref[...]
    wr = wr_ref[...]
    acts = jnp.concatenate(
        [jnp.dot(x, wr[e], preferred_element_type=jnp.float32)
         for e in range(NUM_EXPERTS)], axis=1)
    a2 = acts * acts
    gcols = lax.broadcasted_iota(
        jnp.int32, (NUM_EXPERTS * DIM4ROUTE, NUM_EXPERTS), 0) // DIM4ROUTE
    gids = lax.broadcasted_iota(
        jnp.int32, (NUM_EXPERTS * DIM4ROUTE, NUM_EXPERTS), 1)
    g = (gcols == gids).astype(jnp.float32)
    norms = jnp.sqrt(jnp.dot(a2, g, preferred_element_type=jnp.float32))
    norms_ref[...] = norms

    m = jnp.max(norms, axis=1, keepdims=True)
    p = jnp.exp(norms - m)
    probs = p / jnp.sum(p, axis=1, keepdims=True)

    ids = lax.broadcasted_iota(jnp.int32, (TB, NUM_EXPERTS), 1)
    maxp = jnp.max(probs, axis=1, keepdims=True)
    sel = jnp.min(jnp.where(probs == maxp, ids, NUM_EXPERTS), axis=1)
    oh = (ids == sel[:, None]).astype(jnp.float32)

    tri = (lax.broadcasted_iota(jnp.int32, (TB, TB), 0)
           >= lax.broadcasted_iota(jnp.int32, (TB, TB), 1)).astype(jnp.float32)
    prefix = jnp.dot(tri, oh, preferred_element_type=jnp.float32)
    runc = runc_ref[...]
    rank = (jnp.sum(oh * prefix, axis=1) - 1.0
            + jnp.sum(oh * runc, axis=1))

    sel_ref[...] = sel.reshape(1, 1, TB)
    rank_ref[...] = rank.astype(jnp.int32).reshape(1, 1, TB)

    runc_ref[...] = runc + jnp.sum(oh, axis=0, keepdims=True)
    psum_ref[...] = psum_ref[...] + jnp.sum(probs, axis=0, keepdims=True)

    @pl.when(i == pl.num_programs(0) - 1)
    def _fin():
        counts_ref[...] = runc_ref[...].astype(jnp.int32)
        loss_ref[...] = jnp.sum(
            runc_ref[...] * psum_ref[...], axis=1, keepdims=True) * (
            NUM_EXPERTS / (T * float(T)))


def _routing(hs, w_route):
    nblk = T // TB
    return pl.pallas_call(
        _routing_body,
        grid=(nblk,),
        in_specs=[
            pl.BlockSpec((TB, DIM), lambda i: (i, 0)),
            pl.BlockSpec((NUM_EXPERTS, DIM, DIM4ROUTE), lambda i: (0, 0, 0)),
        ],
        out_specs=[
            pl.BlockSpec((TB, NUM_EXPERTS), lambda i: (i, 0)),
            pl.BlockSpec((1, 1, TB), lambda i: (i, 0, 0)),
            pl.BlockSpec((1, 1, TB), lambda i: (i, 0, 0)),
            pl.BlockSpec((1, NUM_EXPERTS), lambda i: (0, 0)),
            pl.BlockSpec((1, 1), lambda i: (0, 0)),
        ],
        out_shape=[
            jax.ShapeDtypeStruct((T, NUM_EXPERTS), jnp.float32),
            jax.ShapeDtypeStruct((nblk, 1, TB), jnp.int32),
            jax.ShapeDtypeStruct((nblk, 1, TB), jnp.int32),
            jax.ShapeDtypeStruct((1, NUM_EXPERTS), jnp.int32),
            jax.ShapeDtypeStruct((1, 1), jnp.float32),
        ],
        scratch_shapes=[
            pltpu.VMEM((1, NUM_EXPERTS), jnp.float32),
            pltpu.VMEM((1, NUM_EXPERTS), jnp.float32),
        ],
    )(hs, w_route)


def _dispatch_body(hs_hbm, sel_hbm, rank_hbm, start_hbm,
                   hs_sorted_hbm, pos_hbm,
                   start_v, sel_v, rank_v, pos2d, rows, sem):
    wid = lax.axis_index("s") * NC + lax.axis_index("c")
    base = wid * TPW
    pltpu.sync_copy(start_hbm, start_v)
    pltpu.sync_copy(sel_hbm.at[pl.ds(base, TPW)], sel_v)
    pltpu.sync_copy(rank_hbm.at[pl.ds(base, TPW)], rank_v)
    for j in range(TPW // 16):
        s = sel_v[pl.ds(j * 16, 16)]
        r = rank_v[pl.ds(j * 16, 16)]
        st = plsc.load_gather(start_v, [s])
        pos2d[(j * 16) // CH, pl.ds((j * 16) % CH, 16)] = st + r
    pltpu.sync_copy(pos2d, pos_hbm.at[wid])
    for c in range(NCH):
        pltpu.sync_copy(hs_hbm.at[pl.ds(base + c * CH, CH)], rows)
        pltpu.async_copy(rows, hs_sorted_hbm.at[pos2d.at[c]], sem).wait()


def _dispatch(hs, sel, rank, start16):
    mesh = plsc.VectorSubcoreMesh(core_axis_name="c", subcore_axis_name="s")
    fn = functools.partial(
        pl.kernel,
        out_type=[
            jax.ShapeDtypeStruct((PADDED_T, DIM), jnp.float32),
            jax.ShapeDtypeStruct((NW, NCH, CH), jnp.int32),
        ],
        mesh=mesh,
        scratch_types=[
            pltpu.VMEM((16,), jnp.int32),
            pltpu.VMEM((TPW,), jnp.int32),
            pltpu.VMEM((TPW,), jnp.int32),
            pltpu.VMEM((NCH, CH), jnp.int32),
            pltpu.VMEM((CH, DIM), jnp.float32),
            pltpu.SemaphoreType.DMA,
        ],
        compiler_params=pltpu.CompilerParams(needs_layout_passes=False),
    )(_dispatch_body)
    return fn(hs, sel, rank, start16)


HH = HID // 2


def _ffn_compute(x_ref, wr_ref, w3_ref, w1_ref, w2_ref):
    x = x_ref[...]
    a = jnp.dot(x, wr_ref[0], preferred_element_type=jnp.float32)
    h3 = jnp.dot(x, w3_ref[0], preferred_element_type=jnp.float32)
    h1 = jnp.dot(a, w1_ref[0], preferred_element_type=jnp.float32)
    cur = h3 * (h1 * jax.nn.sigmoid(h1))
    return jnp.dot(cur, w2_ref[0], preferred_element_type=jnp.float32)


def _ffn_body_first(te_ref, x_ref, wr_ref, w3_ref, w1_ref, w2_ref, o_ref):
    o_ref[...] = _ffn_compute(x_ref, wr_ref, w3_ref, w1_ref, w2_ref)


def _ffn_body_second(te_ref, x_ref, wr_ref, w3_ref, w1_ref, w2_ref, c_ref,
                     o_ref):
    o_ref[...] = c_ref[...] + _ffn_compute(x_ref, wr_ref, w3_ref, w1_ref,
                                           w2_ref)


def _ffn_half(n_tiles, tile_expert, hs_sorted, w_route, w3, w1, w2, half,
              carry):
    h = half
    in_specs = [
        pl.BlockSpec((BT, DIM), lambda i, te: (i, 0)),
        pl.BlockSpec((1, DIM, DIM4ROUTE), lambda i, te: (te[i], 0, 0)),
        pl.BlockSpec((1, DIM, HH), lambda i, te: (te[i], 0, h)),
        pl.BlockSpec((1, DIM4ROUTE, HH), lambda i, te: (te[i], 0, h)),
        pl.BlockSpec((1, HH, DIM), lambda i, te: (te[i], h, 0)),
    ]
    args = [tile_expert, hs_sorted, w_route, w3, w1, w2]
    body = _ffn_body_first
    if carry is not None:
        in_specs.append(pl.BlockSpec((BT, DIM), lambda i, te: (i, 0)))
        args.append(carry)
        body = _ffn_body_second
    grid_spec = pltpu.PrefetchScalarGridSpec(
        num_scalar_prefetch=1,
        grid=(n_tiles,),
        in_specs=in_specs,
        out_specs=pl.BlockSpec((BT, DIM), lambda i, te: (i, 0)),
    )
    return pl.pallas_call(
        body,
        grid_spec=grid_spec,
        out_shape=jax.ShapeDtypeStruct((PADDED_T, DIM), jnp.float32),
        compiler_params=pltpu.CompilerParams(
            vmem_limit_bytes=100 * 1024 * 1024),
    )(*args)


def _ffn(n_tiles, tile_expert, hs_sorted, w_route, w3, w1, w2):
    part = _ffn_half(n_tiles, tile_expert, hs_sorted, w_route, w3, w1, w2,
                     0, None)
    return _ffn_half(n_tiles, tile_expert, hs_sorted, w_route, w3, w1, w2,
                     1, part)


def _combine_body(src_hbm, pos_hbm, final_hbm, pos2d, rows, sem):
    wid = lax.axis_index("s") * NC + lax.axis_index("c")
    base = wid * TPW
    pltpu.sync_copy(pos_hbm.at[wid], pos2d)
    for c in range(NCH):
        pltpu.async_copy(src_hbm.at[pos2d.at[c]], rows, sem).wait()
        pltpu.sync_copy(rows, final_hbm.at[pl.ds(base + c * CH, CH)])


def _combine(out_sorted, pos):
    mesh = plsc.VectorSubcoreMesh(core_axis_name="c", subcore_axis_name="s")
    fn = functools.partial(
        pl.kernel,
        out_type=jax.ShapeDtypeStruct((T, DIM), jnp.float32),
        mesh=mesh,
        scratch_types=[
            pltpu.VMEM((NCH, CH), jnp.int32),
            pltpu.VMEM((CH, DIM), jnp.float32),
            pltpu.SemaphoreType.DMA,
        ],
    )(_combine_body)
    return fn(out_sorted, pos)


def kernel(hidden_states, W_route, W3, W1, W2):
    b, s, d = hidden_states.shape
    hs = hidden_states.reshape(-1, d)

    norms, sel3, rank3, counts, loss = _routing(hs, W_route)
    sel = sel3.reshape(T)
    rank = rank3.reshape(T)

    cnt = counts[0]
    caps = ((cnt + BT - 1) // BT) * BT
    cum = jnp.cumsum(caps)
    start = cum - caps
    start16 = jnp.concatenate([start, jnp.zeros(8, jnp.int32)])
    total = cum[-1]
    tile_starts = jnp.arange(NUM_TILES, dtype=jnp.int32) * BT
    te = jnp.minimum(jnp.searchsorted(cum, tile_starts, side="right"),
                     NUM_EXPERTS - 1).astype(jnp.int32)
    last_e = jnp.take(te, total // BT - 1)
    tile_expert = jnp.where(tile_starts < total, te, last_e)

    hs_sorted, pos = _dispatch(hs, sel, rank, start16)
    out_sorted = _ffn(total // BT, tile_expert, hs_sorted, W_route, W3, W1, W2)
    final = _combine(out_sorted, pos)

    return (final.reshape(b, s, d), norms, loss.reshape(()))


# speedup vs baseline: 1.0235x; 1.0235x over previous
import functools

import jax
import jax.numpy as jnp
from jax import lax
from jax.experimental import pallas as pl
from jax.experimental.pallas import tpu as pltpu
from jax.experimental.pallas import tpu_sc as plsc

NUM_EXPERTS = 8
DIM = 1024
DIM4ROUTE = 128
HID = 4096
T = 4096
TB = 512
BT = 256
PADDED_T = T + NUM_EXPERTS * BT
NUM_TILES = PADDED_T // BT

NC = 2
NS = 16
NW = NC * NS
TPW = T // NW
CH = 32
NCH = TPW // CH


def _routing_body(x_ref, wr_ref, norms_ref, sel_ref, rank_ref, counts_ref,
                  loss_ref, runc_ref, psum_ref):
    i = pl.program_id(0)

    @pl.when(i == 0)
    def _init():
        runc_ref[...] = jnp.zeros_like(runc_ref)
        psum_ref[...] = jnp.zeros_like(psum_ref)

    x = x_ref[...]
    wr = wr_ref[...]
    cols = []
    for e in range(NUM_EXPERTS):
        a = jnp.dot(x, wr[e], preferred_element_type=jnp.float32)
        cols.append(jnp.sum(a * a, axis=1, keepdims=True))
    norms = jnp.sqrt(jnp.concatenate(cols, axis=1))
    norms_ref[...] = norms

    m = jnp.max(norms, axis=1, keepdims=True)
    p = jnp.exp(norms - m)
    probs = p / jnp.sum(p, axis=1, keepdims=True)

    ids = lax.broadcasted_iota(jnp.int32, (TB, NUM_EXPERTS), 1)
    maxp = jnp.max(probs, axis=1, keepdims=True)
    sel = jnp.min(jnp.where(probs == maxp, ids, NUM_EXPERTS), axis=1)
    oh = (ids == sel[:, None]).astype(jnp.float32)

    tri = (lax.broadcasted_iota(jnp.int32, (TB, TB), 0)
           >= lax.broadcasted_iota(jnp.int32, (TB, TB), 1)).astype(jnp.float32)
    prefix = jnp.dot(tri, oh, preferred_element_type=jnp.float32)
    runc = runc_ref[...]
    rank = (jnp.sum(oh * prefix, axis=1) - 1.0
            + jnp.sum(oh * runc, axis=1))

    sel_ref[...] = sel.reshape(1, 1, TB)
    rank_ref[...] = rank.astype(jnp.int32).reshape(1, 1, TB)

    runc_ref[...] = runc + jnp.sum(oh, axis=0, keepdims=True)
    psum_ref[...] = psum_ref[...] + jnp.sum(probs, axis=0, keepdims=True)

    @pl.when(i == pl.num_programs(0) - 1)
    def _fin():
        counts_ref[...] = runc_ref[...].astype(jnp.int32)
        loss_ref[...] = jnp.sum(
            runc_ref[...] * psum_ref[...], axis=1, keepdims=True) * (
            NUM_EXPERTS / (T * float(T)))


def _routing(hs, w_route):
    nblk = T // TB
    return pl.pallas_call(
        _routing_body,
        grid=(nblk,),
        in_specs=[
            pl.BlockSpec((TB, DIM), lambda i: (i, 0)),
            pl.BlockSpec((NUM_EXPERTS, DIM, DIM4ROUTE), lambda i: (0, 0, 0)),
        ],
        out_specs=[
            pl.BlockSpec((TB, NUM_EXPERTS), lambda i: (i, 0)),
            pl.BlockSpec((1, 1, TB), lambda i: (i, 0, 0)),
            pl.BlockSpec((1, 1, TB), lambda i: (i, 0, 0)),
            pl.BlockSpec((1, NUM_EXPERTS), lambda i: (0, 0)),
            pl.BlockSpec((1, 1), lambda i: (0, 0)),
        ],
        out_shape=[
            jax.ShapeDtypeStruct((T, NUM_EXPERTS), jnp.float32),
            jax.ShapeDtypeStruct((nblk, 1, TB), jnp.int32),
            jax.ShapeDtypeStruct((nblk, 1, TB), jnp.int32),
            jax.ShapeDtypeStruct((1, NUM_EXPERTS), jnp.int32),
            jax.ShapeDtypeStruct((1, 1), jnp.float32),
        ],
        scratch_shapes=[
            pltpu.VMEM((1, NUM_EXPERTS), jnp.float32),
            pltpu.VMEM((1, NUM_EXPERTS), jnp.float32),
        ],
    )(hs, w_route)


def _dispatch_body(hs_hbm, sel_hbm, rank_hbm, start_hbm,
                   hs_sorted_hbm, pos_hbm,
                   start_v, sel_v, rank_v, pos2d, rows0, rows1,
                   isem0, isem1, osem0, osem1):
    wid = lax.axis_index("s") * NC + lax.axis_index("c")
    base = wid * TPW
    bufs = (rows0, rows1)
    isems = (isem0, isem1)
    osems = (osem0, osem1)
    inc = [pltpu.async_copy(hs_hbm.at[pl.ds(base + c * CH, CH)],
                            bufs[c % 2], isems[c % 2]) for c in range(2)]
    pltpu.sync_copy(start_hbm, start_v)
    pltpu.sync_copy(sel_hbm.at[pl.ds(base, TPW)], sel_v)
    pltpu.sync_copy(rank_hbm.at[pl.ds(base, TPW)], rank_v)
    for j in range(TPW // 16):
        s = sel_v[pl.ds(j * 16, 16)]
        r = rank_v[pl.ds(j * 16, 16)]
        st = plsc.load_gather(start_v, [s])
        pos2d[(j * 16) // CH, pl.ds((j * 16) % CH, 16)] = st + r
    pltpu.sync_copy(pos2d, pos_hbm.at[wid])
    scat = [None, None]
    for c in range(NCH):
        b = c % 2
        inc[c].wait()
        scat[b] = pltpu.async_copy(bufs[b], hs_sorted_hbm.at[pos2d.at[c]],
                                   osems[b])
        if c + 2 < NCH:
            scat[b].wait()
            inc.append(pltpu.async_copy(
                hs_hbm.at[pl.ds(base + (c + 2) * CH, CH)], bufs[b],
                isems[b]))
    scat[(NCH - 2) % 2].wait()
    scat[(NCH - 1) % 2].wait()


def _dispatch(hs, sel, rank, start16):
    mesh = plsc.VectorSubcoreMesh(core_axis_name="c", subcore_axis_name="s")
    fn = functools.partial(
        pl.kernel,
        out_type=[
            jax.ShapeDtypeStruct((PADDED_T, DIM), jnp.float32),
            jax.ShapeDtypeStruct((NW, NCH, CH), jnp.int32),
        ],
        mesh=mesh,
        scratch_types=[
            pltpu.VMEM((16,), jnp.int32),
            pltpu.VMEM((TPW,), jnp.int32),
            pltpu.VMEM((TPW,), jnp.int32),
            pltpu.VMEM((NCH, CH), jnp.int32),
            pltpu.VMEM((CH, DIM), jnp.float32),
            pltpu.VMEM((CH, DIM), jnp.float32),
            pltpu.SemaphoreType.DMA,
            pltpu.SemaphoreType.DMA,
            pltpu.SemaphoreType.DMA,
            pltpu.SemaphoreType.DMA,
        ],
        compiler_params=pltpu.CompilerParams(needs_layout_passes=False),
    )(_dispatch_body)
    return fn(hs, sel, rank, start16)


HH = HID // 2


def _ffn_compute(x_ref, wr_ref, w3_ref, w1_ref, w2_ref):
    x = x_ref[...]
    a = jnp.dot(x, wr_ref[0], preferred_element_type=jnp.float32)
    h3 = jnp.dot(x, w3_ref[0], preferred_element_type=jnp.float32)
    h1 = jnp.dot(a, w1_ref[0], preferred_element_type=jnp.float32)
    cur = h3 * (h1 * jax.nn.sigmoid(h1))
    return jnp.dot(cur, w2_ref[0], preferred_element_type=jnp.float32)


def _ffn_body_first(te_ref, x_ref, wr_ref, w3_ref, w1_ref, w2_ref, o_ref):
    o_ref[...] = _ffn_compute(x_ref, wr_ref, w3_ref, w1_ref, w2_ref)


def _ffn_body_second(te_ref, x_ref, wr_ref, w3_ref, w1_ref, w2_ref, c_ref,
                     o_ref):
    o_ref[...] = c_ref[...] + _ffn_compute(x_ref, wr_ref, w3_ref, w1_ref,
                                           w2_ref)


def _ffn_half(n_tiles, tile_expert, hs_sorted, w_route, w3, w1, w2, half,
              carry):
    h = half
    in_specs = [
        pl.BlockSpec((BT, DIM), lambda i, te: (i, 0)),
        pl.BlockSpec((1, DIM, DIM4ROUTE), lambda i, te: (te[i], 0, 0)),
        pl.BlockSpec((1, DIM, HH), lambda i, te: (te[i], 0, h)),
        pl.BlockSpec((1, DIM4ROUTE, HH), lambda i, te: (te[i], 0, h)),
        pl.BlockSpec((1, HH, DIM), lambda i, te: (te[i], h, 0)),
    ]
    args = [tile_expert, hs_sorted, w_route, w3, w1, w2]
    body = _ffn_body_first
    if carry is not None:
        in_specs.append(pl.BlockSpec((BT, DIM), lambda i, te: (i, 0)))
        args.append(carry)
        body = _ffn_body_second
    grid_spec = pltpu.PrefetchScalarGridSpec(
        num_scalar_prefetch=1,
        grid=(n_tiles,),
        in_specs=in_specs,
        out_specs=pl.BlockSpec((BT, DIM), lambda i, te: (i, 0)),
    )
    return pl.pallas_call(
        body,
        grid_spec=grid_spec,
        out_shape=jax.ShapeDtypeStruct((PADDED_T, DIM), jnp.float32),
        compiler_params=pltpu.CompilerParams(
            vmem_limit_bytes=100 * 1024 * 1024),
    )(*args)


def _ffn(n_tiles, tile_expert, hs_sorted, w_route, w3, w1, w2):
    part = _ffn_half(n_tiles, tile_expert, hs_sorted, w_route, w3, w1, w2,
                     0, None)
    return _ffn_half(n_tiles, tile_expert, hs_sorted, w_route, w3, w1, w2,
                     1, part)


def _combine_body(src_hbm, pos_hbm, final_hbm, pos2d, rows0, rows1,
                  isem0, isem1, osem0, osem1):
    wid = lax.axis_index("s") * NC + lax.axis_index("c")
    base = wid * TPW
    bufs = (rows0, rows1)
    isems = (isem0, isem1)
    osems = (osem0, osem1)
    pltpu.sync_copy(pos_hbm.at[wid], pos2d)
    gat = [pltpu.async_copy(src_hbm.at[pos2d.at[c]], bufs[c % 2],
                            isems[c % 2]) for c in range(2)]
    wr = [None, None]
    for c in range(NCH):
        b = c % 2
        gat[c].wait()
        wr[b] = pltpu.async_copy(bufs[b],
                                 final_hbm.at[pl.ds(base + c * CH, CH)],
                                 osems[b])
        if c + 2 < NCH:
            wr[b].wait()
            gat.append(pltpu.async_copy(src_hbm.at[pos2d.at[c + 2]],
                                        bufs[b], isems[b]))
    wr[(NCH - 2) % 2].wait()
    wr[(NCH - 1) % 2].wait()


def _combine(out_sorted, pos):
    mesh = plsc.VectorSubcoreMesh(core_axis_name="c", subcore_axis_name="s")
    fn = functools.partial(
        pl.kernel,
        out_type=jax.ShapeDtypeStruct((T, DIM), jnp.float32),
        mesh=mesh,
        scratch_types=[
            pltpu.VMEM((NCH, CH), jnp.int32),
            pltpu.VMEM((CH, DIM), jnp.float32),
            pltpu.VMEM((CH, DIM), jnp.float32),
            pltpu.SemaphoreType.DMA,
            pltpu.SemaphoreType.DMA,
            pltpu.SemaphoreType.DMA,
            pltpu.SemaphoreType.DMA,
        ],
    )(_combine_body)
    return fn(out_sorted, pos)


def kernel(hidden_states, W_route, W3, W1, W2):
    b, s, d = hidden_states.shape
    hs = hidden_states.reshape(-1, d)

    norms, sel3, rank3, counts, loss = _routing(hs, W_route)
    sel = sel3.reshape(T)
    rank = rank3.reshape(T)

    cnt = counts[0]
    caps = ((cnt + BT - 1) // BT) * BT
    cum = jnp.cumsum(caps)
    start = cum - caps
    start16 = jnp.concatenate([start, jnp.zeros(8, jnp.int32)])
    total = cum[-1]
    tile_starts = jnp.arange(NUM_TILES, dtype=jnp.int32) * BT
    te = jnp.minimum(jnp.searchsorted(cum, tile_starts, side="right"),
                     NUM_EXPERTS - 1).astype(jnp.int32)
    last_e = jnp.take(te, total // BT - 1)
    tile_expert = jnp.where(tile_starts < total, te, last_e)

    hs_sorted, pos = _dispatch(hs, sel, rank, start16)
    out_sorted = _ffn(total // BT, tile_expert, hs_sorted, W_route, W3, W1, W2)
    final = _combine(out_sorted, pos)

    return (final.reshape(b, s, d), norms, loss.reshape(()))


# speedup vs baseline: 1.0242x; 1.0007x over previous
import functools

import jax
import jax.numpy as jnp
from jax import lax
from jax.experimental import pallas as pl
from jax.experimental.pallas import tpu as pltpu
from jax.experimental.pallas import tpu_sc as plsc

NUM_EXPERTS = 8
DIM = 1024
DIM4ROUTE = 128
HID = 4096
T = 4096
TB = 512
BT = 256
PADDED_T = T + NUM_EXPERTS * BT
NUM_TILES = PADDED_T // BT

NC = 2
NS = 16
NW = NC * NS
TPW = T // NW
CH = 32
NCH = TPW // CH


def _routing_body(x_ref, wr_ref, norms_ref, sel_ref, rank_ref, counts_ref,
                  loss_ref, start16_ref, te_ref, runc_ref, psum_ref):
    i = pl.program_id(0)

    @pl.when(i == 0)
    def _init():
        runc_ref[...] = jnp.zeros_like(runc_ref)
        psum_ref[...] = jnp.zeros_like(psum_ref)

    x = x_ref[...]
    wr = wr_ref[...]
    cols = []
    for e in range(NUM_EXPERTS):
        a = jnp.dot(x, wr[e], preferred_element_type=jnp.float32)
        cols.append(jnp.sum(a * a, axis=1, keepdims=True))
    norms = jnp.sqrt(jnp.concatenate(cols, axis=1))
    norms_ref[...] = norms

    m = jnp.max(norms, axis=1, keepdims=True)
    p = jnp.exp(norms - m)
    probs = p / jnp.sum(p, axis=1, keepdims=True)

    ids = lax.broadcasted_iota(jnp.int32, (TB, NUM_EXPERTS), 1)
    maxp = jnp.max(probs, axis=1, keepdims=True)
    sel = jnp.min(jnp.where(probs == maxp, ids, NUM_EXPERTS), axis=1)
    oh = (ids == sel[:, None]).astype(jnp.float32)

    tri = (lax.broadcasted_iota(jnp.int32, (TB, TB), 0)
           >= lax.broadcasted_iota(jnp.int32, (TB, TB), 1)).astype(jnp.float32)
    prefix = jnp.dot(tri, oh, preferred_element_type=jnp.float32)
    runc = runc_ref[...]
    rank = (jnp.sum(oh * prefix, axis=1) - 1.0
            + jnp.sum(oh * runc, axis=1))

    sel_ref[...] = sel.reshape(1, 1, TB)
    rank_ref[...] = rank.astype(jnp.int32).reshape(1, 1, TB)

    runc_ref[...] = runc + jnp.sum(oh, axis=0, keepdims=True)
    psum_ref[...] = psum_ref[...] + jnp.sum(probs, axis=0, keepdims=True)

    @pl.when(i == pl.num_programs(0) - 1)
    def _fin():
        cnt = runc_ref[...]
        counts_ref[...] = cnt.astype(jnp.int32)
        loss_ref[...] = jnp.sum(
            cnt * psum_ref[...], axis=1, keepdims=True) * (
            NUM_EXPERTS / (T * float(T)))
        caps = jnp.ceil(cnt * (1.0 / BT)) * BT
        tri = (lax.broadcasted_iota(jnp.int32, (NUM_EXPERTS, NUM_EXPERTS), 0)
               <= lax.broadcasted_iota(jnp.int32,
                                       (NUM_EXPERTS, NUM_EXPERTS), 1)
               ).astype(jnp.float32)
        cum = jnp.dot(caps, tri, preferred_element_type=jnp.float32)
        start = cum - caps
        ntile = cum[:, NUM_EXPERTS - 1:] * (1.0 / BT)
        start16_ref[...] = jnp.concatenate(
            [start, ntile, jnp.zeros((1, 7), jnp.float32)],
            axis=1).astype(jnp.int32)
        ts = (lax.broadcasted_iota(jnp.int32, (NUM_TILES, 1), 0)
              .astype(jnp.float32) * float(BT))
        te = jnp.sum((cum <= ts).astype(jnp.float32), axis=1, keepdims=True)
        te_ref[...] = jnp.minimum(te, NUM_EXPERTS - 1).astype(jnp.int32)


def _routing(hs, w_route):
    nblk = T // TB
    return pl.pallas_call(
        _routing_body,
        grid=(nblk,),
        in_specs=[
            pl.BlockSpec((TB, DIM), lambda i: (i, 0)),
            pl.BlockSpec((NUM_EXPERTS, DIM, DIM4ROUTE), lambda i: (0, 0, 0)),
        ],
        out_specs=[
            pl.BlockSpec((TB, NUM_EXPERTS), lambda i: (i, 0)),
            pl.BlockSpec((1, 1, TB), lambda i: (i, 0, 0)),
            pl.BlockSpec((1, 1, TB), lambda i: (i, 0, 0)),
            pl.BlockSpec((1, NUM_EXPERTS), lambda i: (0, 0)),
            pl.BlockSpec((1, 1), lambda i: (0, 0)),
            pl.BlockSpec((1, 16), lambda i: (0, 0)),
            pl.BlockSpec((NUM_TILES, 1), lambda i: (0, 0)),
        ],
        out_shape=[
            jax.ShapeDtypeStruct((T, NUM_EXPERTS), jnp.float32),
            jax.ShapeDtypeStruct((nblk, 1, TB), jnp.int32),
            jax.ShapeDtypeStruct((nblk, 1, TB), jnp.int32),
            jax.ShapeDtypeStruct((1, NUM_EXPERTS), jnp.int32),
            jax.ShapeDtypeStruct((1, 1), jnp.float32),
            jax.ShapeDtypeStruct((1, 16), jnp.int32),
            jax.ShapeDtypeStruct((NUM_TILES, 1), jnp.int32),
        ],
        scratch_shapes=[
            pltpu.VMEM((1, NUM_EXPERTS), jnp.float32),
            pltpu.VMEM((1, NUM_EXPERTS), jnp.float32),
        ],
    )(hs, w_route)


def _dispatch_body(hs_hbm, sel_hbm, rank_hbm, start_hbm,
                   hs_sorted_hbm, pos_hbm,
                   start_v, sel_v, rank_v, pos2d, rows0, rows1,
                   isem0, isem1, osem0, osem1):
    wid = lax.axis_index("s") * NC + lax.axis_index("c")
    base = wid * TPW
    bufs = (rows0, rows1)
    isems = (isem0, isem1)
    osems = (osem0, osem1)
    inc = [pltpu.async_copy(hs_hbm.at[pl.ds(base + c * CH, CH)],
                            bufs[c % 2], isems[c % 2]) for c in range(2)]
    pltpu.sync_copy(start_hbm, start_v)
    pltpu.sync_copy(sel_hbm.at[pl.ds(base, TPW)], sel_v)
    pltpu.sync_copy(rank_hbm.at[pl.ds(base, TPW)], rank_v)
    for j in range(TPW // 16):
        s = sel_v[pl.ds(j * 16, 16)]
        r = rank_v[pl.ds(j * 16, 16)]
        st = plsc.load_gather(start_v, [s])
        pos2d[(j * 16) // CH, pl.ds((j * 16) % CH, 16)] = st + r
    pltpu.sync_copy(pos2d, pos_hbm.at[wid])
    scat = [None, None]
    for c in range(NCH):
        b = c % 2
        inc[c].wait()
        scat[b] = pltpu.async_copy(bufs[b], hs_sorted_hbm.at[pos2d.at[c]],
                                   osems[b])
        if c + 2 < NCH:
            scat[b].wait()
            inc.append(pltpu.async_copy(
                hs_hbm.at[pl.ds(base + (c + 2) * CH, CH)], bufs[b],
                isems[b]))
    scat[(NCH - 2) % 2].wait()
    scat[(NCH - 1) % 2].wait()


def _dispatch(hs, sel, rank, start16):
    mesh = plsc.VectorSubcoreMesh(core_axis_name="c", subcore_axis_name="s")
    fn = functools.partial(
        pl.kernel,
        out_type=[
            jax.ShapeDtypeStruct((PADDED_T, DIM), jnp.float32),
            jax.ShapeDtypeStruct((NW, NCH, CH), jnp.int32),
        ],
        mesh=mesh,
        scratch_types=[
            pltpu.VMEM((16,), jnp.int32),
            pltpu.VMEM((TPW,), jnp.int32),
            pltpu.VMEM((TPW,), jnp.int32),
            pltpu.VMEM((NCH, CH), jnp.int32),
            pltpu.VMEM((CH, DIM), jnp.float32),
            pltpu.VMEM((CH, DIM), jnp.float32),
            pltpu.SemaphoreType.DMA,
            pltpu.SemaphoreType.DMA,
            pltpu.SemaphoreType.DMA,
            pltpu.SemaphoreType.DMA,
        ],
        compiler_params=pltpu.CompilerParams(needs_layout_passes=False),
    )(_dispatch_body)
    return fn(hs, sel, rank, start16)


HH = HID // 2


def _ffn_compute(x_ref, wr_ref, w3_ref, w1_ref, w2_ref):
    x = x_ref[...]
    a = jnp.dot(x, wr_ref[0], preferred_element_type=jnp.float32)
    h3 = jnp.dot(x, w3_ref[0], preferred_element_type=jnp.float32)
    h1 = jnp.dot(a, w1_ref[0], preferred_element_type=jnp.float32)
    cur = h3 * (h1 * jax.nn.sigmoid(h1))
    return jnp.dot(cur, w2_ref[0], preferred_element_type=jnp.float32)


def _ffn_body_first(te_ref, x_ref, wr_ref, w3_ref, w1_ref, w2_ref, o_ref):
    o_ref[...] = _ffn_compute(x_ref, wr_ref, w3_ref, w1_ref, w2_ref)


def _ffn_body_second(te_ref, x_ref, wr_ref, w3_ref, w1_ref, w2_ref, c_ref,
                     o_ref):
    o_ref[...] = c_ref[...] + _ffn_compute(x_ref, wr_ref, w3_ref, w1_ref,
                                           w2_ref)


def _ffn_half(n_tiles, tile_expert, hs_sorted, w_route, w3, w1, w2, half,
              carry):
    h = half
    in_specs = [
        pl.BlockSpec((BT, DIM), lambda i, te: (i, 0)),
        pl.BlockSpec((1, DIM, DIM4ROUTE), lambda i, te: (te[i], 0, 0)),
        pl.BlockSpec((1, DIM, HH), lambda i, te: (te[i], 0, h)),
        pl.BlockSpec((1, DIM4ROUTE, HH), lambda i, te: (te[i], 0, h)),
        pl.BlockSpec((1, HH, DIM), lambda i, te: (te[i], h, 0)),
    ]
    args = [tile_expert, hs_sorted, w_route, w3, w1, w2]
    body = _ffn_body_first
    if carry is not None:
        in_specs.append(pl.BlockSpec((BT, DIM), lambda i, te: (i, 0)))
        args.append(carry)
        body = _ffn_body_second
    grid_spec = pltpu.PrefetchScalarGridSpec(
        num_scalar_prefetch=1,
        grid=(n_tiles,),
        in_specs=in_specs,
        out_specs=pl.BlockSpec((BT, DIM), lambda i, te: (i, 0)),
    )
    return pl.pallas_call(
        body,
        grid_spec=grid_spec,
        out_shape=jax.ShapeDtypeStruct((PADDED_T, DIM), jnp.float32),
        compiler_params=pltpu.CompilerParams(
            vmem_limit_bytes=100 * 1024 * 1024),
    )(*args)


def _ffn(n_tiles, tile_expert, hs_sorted, w_route, w3, w1, w2):
    part = _ffn_half(n_tiles, tile_expert, hs_sorted, w_route, w3, w1, w2,
                     0, None)
    return _ffn_half(n_tiles, tile_expert, hs_sorted, w_route, w3, w1, w2,
                     1, part)


def _combine_body(src_hbm, pos_hbm, final_hbm, pos2d, rows0, rows1,
                  isem0, isem1, osem0, osem1):
    wid = lax.axis_index("s") * NC + lax.axis_index("c")
    base = wid * TPW
    bufs = (rows0, rows1)
    isems = (isem0, isem1)
    osems = (osem0, osem1)
    pltpu.sync_copy(pos_hbm.at[wid], pos2d)
    gat = [pltpu.async_copy(src_hbm.at[pos2d.at[c]], bufs[c % 2],
                            isems[c % 2]) for c in range(2)]
    wr = [None, None]
    for c in range(NCH):
        b = c % 2
        gat[c].wait()
        wr[b] = pltpu.async_copy(bufs[b],
                                 final_hbm.at[pl.ds(base + c * CH, CH)],
                                 osems[b])
        if c + 2 < NCH:
            wr[b].wait()
            gat.append(pltpu.async_copy(src_hbm.at[pos2d.at[c + 2]],
                                        bufs[b], isems[b]))
    wr[(NCH - 2) % 2].wait()
    wr[(NCH - 1) % 2].wait()


def _combine(out_sorted, pos):
    mesh = plsc.VectorSubcoreMesh(core_axis_name="c", subcore_axis_name="s")
    fn = functools.partial(
        pl.kernel,
        out_type=jax.ShapeDtypeStruct((T, DIM), jnp.float32),
        mesh=mesh,
        scratch_types=[
            pltpu.VMEM((NCH, CH), jnp.int32),
            pltpu.VMEM((CH, DIM), jnp.float32),
            pltpu.VMEM((CH, DIM), jnp.float32),
            pltpu.SemaphoreType.DMA,
            pltpu.SemaphoreType.DMA,
            pltpu.SemaphoreType.DMA,
            pltpu.SemaphoreType.DMA,
        ],
    )(_combine_body)
    return fn(out_sorted, pos)


def kernel(hidden_states, W_route, W3, W1, W2):
    b, s, d = hidden_states.shape
    hs = hidden_states.reshape(-1, d)

    norms, sel3, rank3, counts, loss, start16, te = _routing(hs, W_route)
    del counts
    sel = sel3.reshape(T)
    rank = rank3.reshape(T)
    s16 = start16.reshape(16)
    n_tiles = s16[8]
    tile_expert = te.reshape(NUM_TILES)

    hs_sorted, pos = _dispatch(hs, sel, rank, s16)
    out_sorted = _ffn(n_tiles, tile_expert, hs_sorted, W_route, W3, W1, W2)
    final = _combine(out_sorted, pos)

    return (final.reshape(b, s, d), norms, loss.reshape(()))
